# Initial kernel scaffold; baseline (speedup 1.0000x reference)
import jax
import jax.numpy as jnp
from jax.experimental import pallas as pl


def kernel(x, params):
    raise NotImplementedError("write your pallas kernel here")


# speedup vs baseline: 9.0573x; 9.0573x over previous
import functools

import jax
import jax.numpy as jnp
import numpy as np
from jax import lax
from jax.experimental import pallas as pl
from jax.experimental.pallas import tpu as pltpu
from jax.experimental.pallas import tpu_sc as plsc

B, N, C = 8, 2000, 128
NP = 2048
RB = 256
EPS = 1e-5
NROW = B * 9 * N
NWORK = 32
CHUNK = 512
NCH = 288
BIG = np.int32(1 << 30)

_f32 = jnp.float32


def _k1(x_ref, w_ref, b_ref, y_ref):
    y_ref[0] = jnp.dot(x_ref[0], w_ref[...], preferred_element_type=_f32) + b_ref[0]


def _k2(y_ref, yT_ref, g_ref, m_ref, d_ref, t_ref, gc_ref, mc_ref, dc_ref,
        tc_ref, xf_ref, idx_ref):
    b = pl.program_id(0)
    xf = jax.nn.relu(g_ref[0] * (y_ref[0] - m_ref[0]) / d_ref[0] + t_ref[0])
    xf_ref[0] = xf
    xfT = jax.nn.relu(gc_ref[...] * (yT_ref[0] - mc_ref[...]) / dc_ref[...]
                      + tc_ref[...])
    xx = jnp.sum(xf * xf, axis=1, keepdims=True)
    xxrow = jnp.sum(xfT * xfT, axis=0, keepdims=True)
    cols = lax.broadcasted_iota(jnp.int32, (RB, NP), 1)
    off = (b * NP).astype(jnp.int32)
    for p in range(NP // RB):
        xfR = xf[p * RB:(p + 1) * RB, :]
        mm = jnp.dot(xfR, xfT, preferred_element_type=_f32)
        vals = (2.0 * mm - xx[p * RB:(p + 1) * RB]) - xxrow
        vals = jnp.where(cols < N, vals, -jnp.inf)
        picks = []
        for _ in range(9):
            mx = jnp.max(vals, axis=1, keepdims=True)
            j = jnp.min(jnp.where(vals == mx, cols, BIG), axis=1, keepdims=True)
            picks.append(j)
            vals = jnp.where(cols == j, -jnp.inf, vals)
        idx_ref[0, pl.ds(p * RB, RB), :] = jnp.concatenate(picks, axis=1) + off


def _sc_gather(table_hbm, idx_hbm, out_hbm, idx_v, rows_v, sem):
    wid = lax.axis_index("s") * 2 + lax.axis_index("c")
    for r0 in range(NCH // NWORK):
        r = wid * (NCH // NWORK) + r0
        pltpu.sync_copy(idx_hbm.at[r], idx_v)
        pltpu.async_copy(table_hbm.at[idx_v], rows_v, sem).wait()
        pltpu.sync_copy(rows_v, out_hbm.at[r])


def _gather_sc(table, idx2d):
    mesh = plsc.VectorSubcoreMesh(core_axis_name="c", subcore_axis_name="s")
    fn = functools.partial(
        pl.kernel,
        mesh=mesh,
        out_type=jax.ShapeDtypeStruct((NCH, CHUNK, C), _f32),
        scratch_types=[
            pltpu.VMEM((CHUNK,), jnp.int32),
            pltpu.VMEM((CHUNK, C), _f32),
            pltpu.SemaphoreType.DMA,
        ],
    )(_sc_gather)
    return fn(table, idx2d)


def _stats(st_ref, s0, ssd):
    st_ref[0] = jnp.concatenate([s0, ssd, jnp.zeros((6, C), _f32)], axis=0)


def _k3(xf_ref, G_ref, wxt_ref, wdt_ref, b_ref, h_ref, st_ref):
    xfb = xf_ref[0]
    term1 = b_ref[0]
    for t in range(3):
        term1 = term1 + jnp.dot(xfb, wxt_ref[t], preferred_element_type=_f32)
    s0 = jnp.zeros((1, C), _f32)
    for s in range(3):
        acc = term1
        for t in range(3):
            d = 2.0 * jnp.minimum(0.0, xfb - G_ref[0, 3 * s + t])
            acc = acc + jnp.dot(d, wdt_ref[t], preferred_element_type=_f32)
        h_ref[0, s] = acc
        s0 = s0 + jnp.sum(acc, axis=0, keepdims=True)
    mb = s0 * _f32(1.0 / (3 * N))
    ssd = jnp.zeros((1, C), _f32)
    for s in range(3):
        dv = h_ref[0, s] - mb
        ssd = ssd + jnp.sum(dv * dv, axis=0, keepdims=True)
    _stats(st_ref, s0, ssd)


def _k4(h_ref, a_ref, c_ref, w_ref, b_ref, z_ref, st_ref):
    hcat = jnp.concatenate(
        [jax.nn.relu(h_ref[0, t] * a_ref[0] + c_ref[0]) for t in range(3)],
        axis=1)
    z = jnp.dot(hcat, w_ref[...], preferred_element_type=_f32) + b_ref[0]
    z_ref[0] = z
    s0 = jnp.sum(z, axis=0, keepdims=True)
    dv = z - s0 * _f32(1.0 / N)
    _stats(st_ref, s0, jnp.sum(dv * dv, axis=0, keepdims=True))


def _k5(z_ref, a_ref, c_ref, w2_ref, b2_ref, od_ref, p4_ref, p4x_ref, st_ref):
    od = jax.nn.relu(z_ref[0] * a_ref[0] + c_ref[0])
    od_ref[0] = od
    xmean = jnp.sum(od, axis=0, keepdims=True) * _f32(1.0 / N)
    pre4 = jnp.dot(od, w2_ref[...], preferred_element_type=_f32) + b2_ref[0]
    pre4x = jnp.dot(xmean, w2_ref[...], preferred_element_type=_f32) + b2_ref[0]
    p4_ref[0] = pre4
    p4x_ref[0] = jnp.concatenate([pre4x, jnp.zeros((7, C), _f32)], axis=0)
    s0 = jnp.sum(pre4, axis=0, keepdims=True) + pre4x
    mb = s0 * _f32(1.0 / (N + 1))
    dv = pre4 - mb
    dx = pre4x - mb
    _stats(st_ref, s0,
           jnp.sum(dv * dv, axis=0, keepdims=True) + dx * dx)


def _k6(od_ref, p4_ref, p4x_ref, a_ref, c_ref, cw_ref, cb_ref, sw_ref, sb_ref,
        w2_ref, b2_ref, m0_ref, mbd0_ref, mbd1_ref, b30_ref, b31_ref, gg_ref,
        tg_ref, wca_ref, bca_ref, p5_ref, st_ref):
    b = pl.program_id(0)
    od = od_ref[0]
    yn = jax.nn.relu(p4_ref[0] * a_ref[0] + c_ref[0])
    ynx = jax.nn.relu(p4x_ref[0, 0:1, :] * a_ref[0] + c_ref[0])
    xh2 = cw_ref[0] * yn + cb_ref[0]
    xw2 = sw_ref[0] * ynx + sb_ref[0]
    att_h = jax.nn.sigmoid(
        jnp.dot(xh2, w2_ref[...], preferred_element_type=_f32) + b2_ref[0])
    att_w = jax.nn.sigmoid(
        jnp.dot(xw2, w2_ref[...], preferred_element_type=_f32) + b2_ref[0])
    out1 = od * att_h * att_w
    m0 = m0_ref[0]
    mean_all = jnp.sum(out1, axis=0, keepdims=True) * _f32(1.0 / N)
    logit0 = jnp.dot(mean_all * m0, mbd0_ref[...],
                     preferred_element_type=_f32) + b30_ref[0]
    sig0 = jax.nn.sigmoid(logit0)
    dv0 = out1 - mean_all
    va = jnp.sum(dv0 * dv0, axis=0, keepdims=True) * _f32(1.0 / N)
    xsn = dv0 / jnp.sqrt(va + EPS) * gg_ref[0] + tg_ref[0]
    logit1 = jnp.dot(xsn * (1.0 - m0), mbd1_ref[...],
                     preferred_element_type=_f32) + b31_ref[0]
    sig1 = jax.nn.sigmoid(logit1)
    comb = out1 * (m0 * sig0 + (1.0 - m0) * sig1)
    t1 = jnp.dot(comb, wca_ref[...], preferred_element_type=_f32) + bca_ref[0]
    mu1 = jnp.sum(t1, axis=0, keepdims=True) * _f32(1.0 / N)
    dv1 = t1 - mu1
    va1 = jnp.sum(dv1 * dv1, axis=0, keepdims=True) * _f32(1.0 / N)
    t1n = dv1 / jnp.sqrt(va1 + EPS)
    p5_ref[0] = t1n
    s0 = jnp.sum(t1n, axis=0, keepdims=True)
    dv = t1n - s0 * _f32(1.0 / N)
    _stats(st_ref, s0, jnp.sum(dv * dv, axis=0, keepdims=True))


def _k7(p5_ref, a_ref, c_ref, w_ref, b_ref, p6_ref, st_ref):
    h1 = jax.nn.relu(p5_ref[0] * a_ref[0] + c_ref[0])
    h2 = jnp.dot(h1, w_ref[...], preferred_element_type=_f32) + b_ref[0]
    mu = jnp.sum(h2, axis=0, keepdims=True) * _f32(1.0 / N)
    dv0 = h2 - mu
    va = jnp.sum(dv0 * dv0, axis=0, keepdims=True) * _f32(1.0 / N)
    h2n = dv0 / jnp.sqrt(va + EPS)
    p6_ref[0] = h2n
    s0 = jnp.sum(h2n, axis=0, keepdims=True)
    dv = h2n - s0 * _f32(1.0 / N)
    _stats(st_ref, s0, jnp.sum(dv * dv, axis=0, keepdims=True))


def _bf(v):
    return v.astype(jnp.bfloat16).astype(_f32)


def _k8(p6_ref, a_ref, c_ref, od_ref, ww_ref, bw_ref, wg_ref, bg_ref,
        ex_ref, p7_ref, st_ref):
    b = pl.program_id(0)
    t3 = p6_ref[0] * a_ref[0] + c_ref[0]
    ex = jax.nn.relu(t3 + od_ref[0])
    ex_ref[0] = ex
    wv = jnp.dot(ex, ww_ref[...], preferred_element_type=_f32) \
        + bw_ref[0, 0:1, 0:1]
    w2 = _bf(jax.nn.relu(jnp.tanh(wv)))
    s = jnp.sum(w2 * w2, axis=0, keepdims=True)[0:1, 0:1]
    dd = jnp.sqrt(1.0 / (_f32(N) * s + 1.0))
    db = _bf(dd)
    qo = _bf(_bf(db * _bf(s)) * db)
    qd = _bf(_bf(db * _bf(s + 1.0)) * db)
    exb = _bf(ex)
    colsum = jnp.sum(exb, axis=0, keepdims=True)
    u = qo * colsum + (qd - qo) * exb
    pre7 = jnp.dot(u, wg_ref[...], preferred_element_type=_f32) + bg_ref[0]
    p7_ref[0] = pre7
    s0 = jnp.sum(pre7, axis=0, keepdims=True)
    dv = pre7 - s0 * _f32(1.0 / N)
    _stats(st_ref, s0, jnp.sum(dv * dv, axis=0, keepdims=True))


def _k9(p7_ref, a_ref, c_ref, ex_ref, wf_ref, bf_ref, lg_ref):
    fin = jax.nn.relu(p7_ref[0] * a_ref[0] + c_ref[0]) + ex_ref[0]
    lg_ref[0] = jnp.dot(fin, wf_ref[...], preferred_element_type=_f32) \
        + bf_ref[0, 0:1, 0:1]


def _vspec(shape):
    nd = len(shape)
    return pl.BlockSpec(shape, lambda b, _nd=nd: (0,) * _nd)


def _bspec(shape):
    nd = len(shape)
    return pl.BlockSpec(shape, lambda b, _nd=nd: (b,) + (0,) * (_nd - 1))


_STAT_SPEC = pl.BlockSpec((1, 8, C), lambda b: (b, 0, 0))
_STAT_SHAPE = jax.ShapeDtypeStruct((B, 8, C), jnp.float32)


def _fold2(st, n_b, g, t):
    s0 = st[:, 0, :]
    ssd = st[:, 1, :]
    mb = s0 / n_b
    n = B * n_b
    m = jnp.sum(s0, axis=0) / n
    v = (jnp.sum(ssd, axis=0) + n_b * jnp.sum((mb - m) ** 2, axis=0)) / n
    a = g / jnp.sqrt(v + EPS)
    return a.reshape(1, C), (t - m * a).reshape(1, C)


def kernel(x, params):
    p = params
    xin = x[:, 0]
    xpad = jnp.concatenate(
        [xin, jnp.zeros((B, NP - N, 4), _f32)], axis=1)

    y = pl.pallas_call(
        _k1,
        grid=(B,),
        in_specs=[_bspec((1, NP, 4)), _vspec((4, C)), _vspec((1, C))],
        out_specs=_bspec((1, NP, C)),
        out_shape=jax.ShapeDtypeStruct((B, NP, C), _f32),
    )(xpad, p['Wci'][:, :, 0, 0].T, p['bci'].reshape(1, C))

    yv = y[:, :N, :]
    m1 = jnp.mean(yv, axis=(0, 1))
    v1 = jnp.var(yv, axis=(0, 1))
    d1 = jnp.sqrt(v1 + EPS)
    yT = jnp.transpose(y, (0, 2, 1))

    row = lambda a: a.reshape(1, C)
    col = lambda a: a.reshape(C, 1)

    xf, idx = pl.pallas_call(
        _k2,
        grid=(B,),
        in_specs=[_bspec((1, NP, C)), _bspec((1, C, NP)),
                  _vspec((1, C)), _vspec((1, C)), _vspec((1, C)), _vspec((1, C)),
                  _vspec((C, 1)), _vspec((C, 1)), _vspec((C, 1)), _vspec((C, 1))],
        out_specs=[_bspec((1, NP, C)), _bspec((1, NP, 9))],
        out_shape=[jax.ShapeDtypeStruct((B, NP, C), _f32),
                   jax.ShapeDtypeStruct((B, NP, 9), jnp.int32)],
    )(y, yT, row(p['gci']), row(m1), row(d1), row(p['tci']),
      col(p['gci']), col(m1), col(d1), col(p['tci']))

    idx_t = jnp.transpose(idx[:, :N, :], (0, 2, 1)).reshape(-1)
    idx2d = jnp.concatenate(
        [idx_t, jnp.zeros((NCH * CHUNK - NROW,), jnp.int32)]).reshape(NCH, CHUNK)
    table = xf.reshape(B * NP, C)
    rows = _gather_sc(table, idx2d)
    G = rows.reshape(-1, C)[:NROW].reshape(B, 9, N, C)

    xfu = xf[:, :N, :]

    Wd1 = p['Wd1']
    wxt = jnp.stack([Wd1[:, :C, 0, t].T for t in range(3)], axis=0)
    wdt = jnp.stack([Wd1[:, C:, 0, t].T for t in range(3)], axis=0)
    h, st2 = pl.pallas_call(
        _k3,
        grid=(B,),
        in_specs=[_bspec((1, N, C)), _bspec((1, 9, N, C)),
                  _vspec((3, C, C)), _vspec((3, C, C)), _vspec((1, C))],
        out_specs=[_bspec((1, 3, N, C)), _STAT_SPEC],
        out_shape=[jax.ShapeDtypeStruct((B, 3, N, C), _f32), _STAT_SHAPE],
    )(xfu, G, wxt, wdt, row(p['bd1']))
    a2, c2 = _fold2(st2, N * 3, p['gd1'], p['td1'])

    w2flat = jnp.concatenate([p['Wd2'][:, :, 0, t].T for t in range(3)], axis=0)
    z, st3 = pl.pallas_call(
        _k4,
        grid=(B,),
        in_specs=[_bspec((1, 3, N, C)), _vspec((1, C)), _vspec((1, C)),
                  _vspec((3 * C, C)), _vspec((1, C))],
        out_specs=[_bspec((1, N, C)), _STAT_SPEC],
        out_shape=[jax.ShapeDtypeStruct((B, N, C), _f32), _STAT_SHAPE],
    )(h, a2, c2, w2flat, row(p['bd2']))
    a3, c3 = _fold2(st3, N, p['gd2'], p['td2'])

    w2t = p['W2'][:, :, 0, 0].T
    od, p4, p4x, st4 = pl.pallas_call(
        _k5,
        grid=(B,),
        in_specs=[_bspec((1, N, C)), _vspec((1, C)), _vspec((1, C)),
                  _vspec((C, C)), _vspec((1, C))],
        out_specs=[_bspec((1, N, C)), _bspec((1, N, C)), _bspec((1, 8, C)),
                   _STAT_SPEC],
        out_shape=[jax.ShapeDtypeStruct((B, N, C), _f32),
                   jax.ShapeDtypeStruct((B, N, C), _f32),
                   jax.ShapeDtypeStruct((B, 8, C), _f32), _STAT_SHAPE],
    )(z, a3, c3, w2t, row(p['b2']))
    a4, c4 = _fold2(st4, N + 1, p['g_bn1'], p['t_bn1'])

    P0 = np.zeros((8, C, 8), np.float32)
    P1 = np.zeros((8, C, 8), np.float32)
    for g in range(8):
        for i in range(8):
            P0[g, 16 * g + i, i] = 1.0
            P1[g, 16 * g + 8 + i, i] = 1.0
    P0 = jnp.asarray(P0)
    P1 = jnp.asarray(P1)
    W3t = p['W3'][:, :, 0, 0].T
    mbd0 = jnp.einsum('gai,gbj,ij->ab', P0, P0, W3t)
    mbd1 = jnp.einsum('gai,gbj,ij->ab', P1, P1, W3t)
    b30 = jnp.einsum('gaj,j->a', P0, p['b3']).reshape(1, C)
    b31 = jnp.einsum('gaj,j->a', P1, p['b3']).reshape(1, C)
    ggl = jnp.einsum('gaj,j->a', P1, p['g_gn']).reshape(1, C)
    tgl = jnp.einsum('gaj,j->a', P1, p['t_gn']).reshape(1, C)
    m0 = jnp.asarray(
        np.where((np.arange(C) % 16) < 8, 1.0, 0.0).astype(np.float32)
    ).reshape(1, C)
    cold = np.arange(C)
    cnew = (cold % 64) * 2 + cold // 64
    wca = p['Wc1a'][:, :, 0, 0].T[jnp.asarray(cnew), :]

    p5, st5 = pl.pallas_call(
        _k6,
        grid=(B,),
        in_specs=[_bspec((1, N, C)), _bspec((1, N, C)), _bspec((1, 8, C)),
                  _vspec((1, C)), _vspec((1, C)), _vspec((1, C)), _vspec((1, C)),
                  _vspec((1, C)), _vspec((1, C)), _vspec((C, C)), _vspec((1, C)),
                  _vspec((1, C)), _vspec((C, C)), _vspec((C, C)), _vspec((1, C)),
                  _vspec((1, C)), _vspec((1, C)), _vspec((1, C)), _vspec((C, C)),
                  _vspec((1, C))],
        out_specs=[_bspec((1, N, C)), _STAT_SPEC],
        out_shape=[jax.ShapeDtypeStruct((B, N, C), _f32), _STAT_SHAPE],
    )(od, p4, p4x, a4, c4,
      row(p['cweight1'][0, :, 0, 0]), row(p['cbias1'][0, :, 0, 0]),
      row(p['sweight2'][0, :, 0, 0]), row(p['sbias2'][0, :, 0, 0]),
      w2t, row(p['b2']), m0, mbd0, mbd1, b30, b31, ggl, tgl,
      wca, row(p['bc1a']))
    a5, c5 = _fold2(st5, N, p['gc1a'], p['tc1a'])

    p6, st6 = pl.pallas_call(
        _k7,
        grid=(B,),
        in_specs=[_bspec((1, N, C)), _vspec((1, C)), _vspec((1, C)),
                  _vspec((C, C)), _vspec((1, C))],
        out_specs=[_bspec((1, N, C)), _STAT_SPEC],
        out_shape=[jax.ShapeDtypeStruct((B, N, C), _f32), _STAT_SHAPE],
    )(p5, a5, c5, p['Wc1b'][:, :, 0, 0].T, row(p['bc1b']))
    a6, c6 = _fold2(st6, N, p['gc1b'], p['tc1b'])

    bw_arr = jnp.full((1, 8, C), p['bw'][0], _f32)
    ex, p7, st7 = pl.pallas_call(
        _k8,
        grid=(B,),
        in_specs=[_bspec((1, N, C)), _vspec((1, C)), _vspec((1, C)),
                  _bspec((1, N, C)), _vspec((C, 1)), _vspec((1, 8, C)),
                  _vspec((C, C)), _vspec((1, C))],
        out_specs=[_bspec((1, N, C)), _bspec((1, N, C)), _STAT_SPEC],
        out_shape=[jax.ShapeDtypeStruct((B, N, C), _f32),
                   jax.ShapeDtypeStruct((B, N, C), _f32), _STAT_SHAPE],
    )(p6, a6, c6, od, p['Ww'][0, :, 0, 0].reshape(C, 1), bw_arr,
      p['Wg'][:, :, 0, 0].T, row(p['bg']))
    a7, c7 = _fold2(st7, N, p['gg'], p['tg'])

    bf_arr = jnp.full((1, 8, C), p['bf'][0], _f32)
    lg = pl.pallas_call(
        _k9,
        grid=(B,),
        in_specs=[_bspec((1, N, C)), _vspec((1, C)), _vspec((1, C)),
                  _bspec((1, N, C)), _vspec((C, 1)), _vspec((1, 8, C))],
        out_specs=_bspec((1, N, 1)),
        out_shape=jax.ShapeDtypeStruct((B, N, 1), _f32),
    )(p7, a7, c7, ex, p['Wf'][0, :, 0, 0].reshape(C, 1), bf_arr)

    return lg[:, :, 0]


# speedup vs baseline: 9.2077x; 1.0166x over previous
import functools

import jax
import jax.numpy as jnp
import numpy as np
from jax import lax
from jax.experimental import pallas as pl
from jax.experimental.pallas import tpu as pltpu
from jax.experimental.pallas import tpu_sc as plsc

B, N, C = 8, 2000, 128
NP = 2048
RB = 256
EPS = 1e-5
NROW = B * 9 * N
NWORK = 32
CHUNK = 384
NCH = 384
BIG = np.int32(1 << 30)

_f32 = jnp.float32


def _k1(x_ref, w_ref, b_ref, y_ref):
    y_ref[0] = jnp.dot(x_ref[0], w_ref[...], preferred_element_type=_f32) + b_ref[0]


def _k2(y_ref, yT_ref, g_ref, m_ref, d_ref, t_ref, gc_ref, mc_ref, dc_ref,
        tc_ref, xf_ref, idx_ref):
    b = pl.program_id(0)
    xf = jax.nn.relu(g_ref[0] * (y_ref[0] - m_ref[0]) / d_ref[0] + t_ref[0])
    xf_ref[0] = xf
    xfT = jax.nn.relu(gc_ref[...] * (yT_ref[0] - mc_ref[...]) / dc_ref[...]
                      + tc_ref[...])
    xx = jnp.sum(xf * xf, axis=1, keepdims=True)
    xxrow = jnp.sum(xfT * xfT, axis=0, keepdims=True)
    cols = lax.broadcasted_iota(jnp.int32, (RB, NP), 1)
    off = (b * NP).astype(jnp.int32)
    for p in range(NP // RB):
        xfR = xf[p * RB:(p + 1) * RB, :]
        mm = jnp.dot(xfR, xfT, preferred_element_type=_f32)
        vals = (2.0 * mm - xx[p * RB:(p + 1) * RB]) - xxrow
        vals = jnp.where(cols < N, vals, -jnp.inf)
        picks = []
        for _ in range(9):
            mx = jnp.max(vals, axis=1, keepdims=True)
            j = jnp.min(jnp.where(vals == mx, cols, BIG), axis=1, keepdims=True)
            picks.append(j)
            vals = jnp.where(cols == j, -jnp.inf, vals)
        idx_ref[0, pl.ds(p * RB, RB), :] = jnp.concatenate(picks, axis=1) + off


def _sc_gather(table_hbm, idx_hbm, out_hbm, idx0, idx1, rows0, rows1,
               sg0, sg1, so0, so1):
    wid = lax.axis_index("s") * 2 + lax.axis_index("c")
    nch = NCH // NWORK
    base = wid * nch
    idxb = [idx0, idx1]
    rowsb = [rows0, rows1]
    sg = [sg0, sg1]
    so = [so0, so1]
    for r0 in range(nch):
        p = r0 % 2
        if r0 >= 2:
            pltpu.make_async_copy(rowsb[p], out_hbm.at[base + r0 - 2],
                                  so[p]).wait()
        pltpu.sync_copy(idx_hbm.at[base + r0], idxb[p])
        pltpu.async_copy(table_hbm.at[idxb[p]], rowsb[p], sg[p])
        if r0 >= 1:
            q = (r0 - 1) % 2
            pltpu.make_async_copy(table_hbm.at[idxb[q]], rowsb[q], sg[q]).wait()
            pltpu.async_copy(rowsb[q], out_hbm.at[base + r0 - 1], so[q])
    pl2 = (nch - 1) % 2
    pltpu.make_async_copy(table_hbm.at[idxb[pl2]], rowsb[pl2], sg[pl2]).wait()
    pltpu.async_copy(rowsb[pl2], out_hbm.at[base + nch - 1], so[pl2])
    pltpu.make_async_copy(rowsb[1 - pl2], out_hbm.at[base + nch - 2],
                          so[1 - pl2]).wait()
    pltpu.make_async_copy(rowsb[pl2], out_hbm.at[base + nch - 1],
                          so[pl2]).wait()


def _gather_sc(table, idx2d):
    mesh = plsc.VectorSubcoreMesh(core_axis_name="c", subcore_axis_name="s")
    fn = functools.partial(
        pl.kernel,
        mesh=mesh,
        out_type=jax.ShapeDtypeStruct((NCH, CHUNK, C), _f32),
        scratch_types=[
            pltpu.VMEM((CHUNK,), jnp.int32),
            pltpu.VMEM((CHUNK,), jnp.int32),
            pltpu.VMEM((CHUNK, C), _f32),
            pltpu.VMEM((CHUNK, C), _f32),
            pltpu.SemaphoreType.DMA,
            pltpu.SemaphoreType.DMA,
            pltpu.SemaphoreType.DMA,
            pltpu.SemaphoreType.DMA,
        ],
    )(_sc_gather)
    return fn(table, idx2d)


def _stats(st_ref, s0, ssd):
    st_ref[0] = jnp.concatenate([s0, ssd, jnp.zeros((6, C), _f32)], axis=0)


def _k3(xf_ref, G_ref, wxt_ref, wdt_ref, b_ref, h_ref, st_ref):
    xfb = xf_ref[0]
    term1 = b_ref[0]
    for t in range(3):
        term1 = term1 + jnp.dot(xfb, wxt_ref[t], preferred_element_type=_f32)
    s0 = jnp.zeros((1, C), _f32)
    for s in range(3):
        acc = term1
        for t in range(3):
            d = 2.0 * jnp.minimum(0.0, xfb - G_ref[0, 3 * s + t])
            acc = acc + jnp.dot(d, wdt_ref[t], preferred_element_type=_f32)
        h_ref[0, s] = acc
        s0 = s0 + jnp.sum(acc, axis=0, keepdims=True)
    mb = s0 * _f32(1.0 / (3 * N))
    ssd = jnp.zeros((1, C), _f32)
    for s in range(3):
        dv = h_ref[0, s] - mb
        ssd = ssd + jnp.sum(dv * dv, axis=0, keepdims=True)
    _stats(st_ref, s0, ssd)


def _k4(h_ref, a_ref, c_ref, w_ref, b_ref, z_ref, st_ref):
    hcat = jnp.concatenate(
        [jax.nn.relu(h_ref[0, t] * a_ref[0] + c_ref[0]) for t in range(3)],
        axis=1)
    z = jnp.dot(hcat, w_ref[...], preferred_element_type=_f32) + b_ref[0]
    z_ref[0] = z
    s0 = jnp.sum(z, axis=0, keepdims=True)
    dv = z - s0 * _f32(1.0 / N)
    _stats(st_ref, s0, jnp.sum(dv * dv, axis=0, keepdims=True))


def _k5(z_ref, a_ref, c_ref, w2_ref, b2_ref, od_ref, p4_ref, p4x_ref, st_ref):
    od = jax.nn.relu(z_ref[0] * a_ref[0] + c_ref[0])
    od_ref[0] = od
    xmean = jnp.sum(od, axis=0, keepdims=True) * _f32(1.0 / N)
    pre4 = jnp.dot(od, w2_ref[...], preferred_element_type=_f32) + b2_ref[0]
    pre4x = jnp.dot(xmean, w2_ref[...], preferred_element_type=_f32) + b2_ref[0]
    p4_ref[0] = pre4
    p4x_ref[0] = jnp.concatenate([pre4x, jnp.zeros((7, C), _f32)], axis=0)
    s0 = jnp.sum(pre4, axis=0, keepdims=True) + pre4x
    mb = s0 * _f32(1.0 / (N + 1))
    dv = pre4 - mb
    dx = pre4x - mb
    _stats(st_ref, s0,
           jnp.sum(dv * dv, axis=0, keepdims=True) + dx * dx)


def _k6(od_ref, p4_ref, p4x_ref, a_ref, c_ref, cw_ref, cb_ref, sw_ref, sb_ref,
        w2_ref, b2_ref, m0_ref, mbd0_ref, mbd1_ref, b30_ref, b31_ref, gg_ref,
        tg_ref, wca_ref, bca_ref, p5_ref, st_ref):
    b = pl.program_id(0)
    od = od_ref[0]
    yn = jax.nn.relu(p4_ref[0] * a_ref[0] + c_ref[0])
    ynx = jax.nn.relu(p4x_ref[0, 0:1, :] * a_ref[0] + c_ref[0])
    xh2 = cw_ref[0] * yn + cb_ref[0]
    xw2 = sw_ref[0] * ynx + sb_ref[0]
    att_h = jax.nn.sigmoid(
        jnp.dot(xh2, w2_ref[...], preferred_element_type=_f32) + b2_ref[0])
    att_w = jax.nn.sigmoid(
        jnp.dot(xw2, w2_ref[...], preferred_element_type=_f32) + b2_ref[0])
    out1 = od * att_h * att_w
    m0 = m0_ref[0]
    mean_all = jnp.sum(out1, axis=0, keepdims=True) * _f32(1.0 / N)
    logit0 = jnp.dot(mean_all * m0, mbd0_ref[...],
                     preferred_element_type=_f32) + b30_ref[0]
    sig0 = jax.nn.sigmoid(logit0)
    dv0 = out1 - mean_all
    va = jnp.sum(dv0 * dv0, axis=0, keepdims=True) * _f32(1.0 / N)
    xsn = dv0 / jnp.sqrt(va + EPS) * gg_ref[0] + tg_ref[0]
    logit1 = jnp.dot(xsn * (1.0 - m0), mbd1_ref[...],
                     preferred_element_type=_f32) + b31_ref[0]
    sig1 = jax.nn.sigmoid(logit1)
    comb = out1 * (m0 * sig0 + (1.0 - m0) * sig1)
    t1 = jnp.dot(comb, wca_ref[...], preferred_element_type=_f32) + bca_ref[0]
    mu1 = jnp.sum(t1, axis=0, keepdims=True) * _f32(1.0 / N)
    dv1 = t1 - mu1
    va1 = jnp.sum(dv1 * dv1, axis=0, keepdims=True) * _f32(1.0 / N)
    t1n = dv1 / jnp.sqrt(va1 + EPS)
    p5_ref[0] = t1n
    s0 = jnp.sum(t1n, axis=0, keepdims=True)
    dv = t1n - s0 * _f32(1.0 / N)
    _stats(st_ref, s0, jnp.sum(dv * dv, axis=0, keepdims=True))


def _k7(p5_ref, a_ref, c_ref, w_ref, b_ref, p6_ref, st_ref):
    h1 = jax.nn.relu(p5_ref[0] * a_ref[0] + c_ref[0])
    h2 = jnp.dot(h1, w_ref[...], preferred_element_type=_f32) + b_ref[0]
    mu = jnp.sum(h2, axis=0, keepdims=True) * _f32(1.0 / N)
    dv0 = h2 - mu
    va = jnp.sum(dv0 * dv0, axis=0, keepdims=True) * _f32(1.0 / N)
    h2n = dv0 / jnp.sqrt(va + EPS)
    p6_ref[0] = h2n
    s0 = jnp.sum(h2n, axis=0, keepdims=True)
    dv = h2n - s0 * _f32(1.0 / N)
    _stats(st_ref, s0, jnp.sum(dv * dv, axis=0, keepdims=True))


def _bf(v):
    return v.astype(jnp.bfloat16).astype(_f32)


def _k8(p6_ref, a_ref, c_ref, od_ref, ww_ref, bw_ref, wg_ref, bg_ref,
        ex_ref, p7_ref, st_ref):
    b = pl.program_id(0)
    t3 = p6_ref[0] * a_ref[0] + c_ref[0]
    ex = jax.nn.relu(t3 + od_ref[0])
    ex_ref[0] = ex
    wv = jnp.dot(ex, ww_ref[...], preferred_element_type=_f32) \
        + bw_ref[0, 0:1, 0:1]
    w2 = _bf(jax.nn.relu(jnp.tanh(wv)))
    s = jnp.sum(w2 * w2, axis=0, keepdims=True)[0:1, 0:1]
    dd = jnp.sqrt(1.0 / (_f32(N) * s + 1.0))
    db = _bf(dd)
    qo = _bf(_bf(db * _bf(s)) * db)
    qd = _bf(_bf(db * _bf(s + 1.0)) * db)
    exb = _bf(ex)
    colsum = jnp.sum(exb, axis=0, keepdims=True)
    u = qo * colsum + (qd - qo) * exb
    pre7 = jnp.dot(u, wg_ref[...], preferred_element_type=_f32) + bg_ref[0]
    p7_ref[0] = pre7
    s0 = jnp.sum(pre7, axis=0, keepdims=True)
    dv = pre7 - s0 * _f32(1.0 / N)
    _stats(st_ref, s0, jnp.sum(dv * dv, axis=0, keepdims=True))


def _k9(p7_ref, a_ref, c_ref, ex_ref, wf_ref, bf_ref, lg_ref):
    fin = jax.nn.relu(p7_ref[0] * a_ref[0] + c_ref[0]) + ex_ref[0]
    lg_ref[0] = jnp.dot(fin, wf_ref[...], preferred_element_type=_f32) \
        + bf_ref[0, 0:1, 0:1]


def _vspec(shape):
    nd = len(shape)
    return pl.BlockSpec(shape, lambda b, _nd=nd: (0,) * _nd)


def _bspec(shape):
    nd = len(shape)
    return pl.BlockSpec(shape, lambda b, _nd=nd: (b,) + (0,) * (_nd - 1))


_STAT_SPEC = pl.BlockSpec((1, 8, C), lambda b: (b, 0, 0))
_STAT_SHAPE = jax.ShapeDtypeStruct((B, 8, C), jnp.float32)


def _fold2(st, n_b, g, t):
    s0 = st[:, 0, :]
    ssd = st[:, 1, :]
    mb = s0 / n_b
    n = B * n_b
    m = jnp.sum(s0, axis=0) / n
    v = (jnp.sum(ssd, axis=0) + n_b * jnp.sum((mb - m) ** 2, axis=0)) / n
    a = g / jnp.sqrt(v + EPS)
    return a.reshape(1, C), (t - m * a).reshape(1, C)


def kernel(x, params):
    p = params
    xin = x[:, 0]
    xpad = jnp.concatenate(
        [xin, jnp.zeros((B, NP - N, 4), _f32)], axis=1)

    y = pl.pallas_call(
        _k1,
        grid=(B,),
        in_specs=[_bspec((1, NP, 4)), _vspec((4, C)), _vspec((1, C))],
        out_specs=_bspec((1, NP, C)),
        out_shape=jax.ShapeDtypeStruct((B, NP, C), _f32),
    )(xpad, p['Wci'][:, :, 0, 0].T, p['bci'].reshape(1, C))

    yv = y[:, :N, :]
    m1 = jnp.mean(yv, axis=(0, 1))
    v1 = jnp.var(yv, axis=(0, 1))
    d1 = jnp.sqrt(v1 + EPS)
    yT = jnp.transpose(y, (0, 2, 1))

    row = lambda a: a.reshape(1, C)
    col = lambda a: a.reshape(C, 1)

    xf, idx = pl.pallas_call(
        _k2,
        grid=(B,),
        in_specs=[_bspec((1, NP, C)), _bspec((1, C, NP)),
                  _vspec((1, C)), _vspec((1, C)), _vspec((1, C)), _vspec((1, C)),
                  _vspec((C, 1)), _vspec((C, 1)), _vspec((C, 1)), _vspec((C, 1))],
        out_specs=[_bspec((1, NP, C)), _bspec((1, NP, 9))],
        out_shape=[jax.ShapeDtypeStruct((B, NP, C), _f32),
                   jax.ShapeDtypeStruct((B, NP, 9), jnp.int32)],
    )(y, yT, row(p['gci']), row(m1), row(d1), row(p['tci']),
      col(p['gci']), col(m1), col(d1), col(p['tci']))

    idx_t = jnp.transpose(idx[:, :N, :], (0, 2, 1)).reshape(-1)
    idx2d = jnp.concatenate(
        [idx_t, jnp.zeros((NCH * CHUNK - NROW,), jnp.int32)]).reshape(NCH, CHUNK)
    table = xf.reshape(B * NP, C)
    rows = _gather_sc(table, idx2d)
    G = rows.reshape(-1, C)[:NROW].reshape(B, 9, N, C)

    xfu = xf[:, :N, :]

    Wd1 = p['Wd1']
    wxt = jnp.stack([Wd1[:, :C, 0, t].T for t in range(3)], axis=0)
    wdt = jnp.stack([Wd1[:, C:, 0, t].T for t in range(3)], axis=0)
    h, st2 = pl.pallas_call(
        _k3,
        grid=(B,),
        in_specs=[_bspec((1, N, C)), _bspec((1, 9, N, C)),
                  _vspec((3, C, C)), _vspec((3, C, C)), _vspec((1, C))],
        out_specs=[_bspec((1, 3, N, C)), _STAT_SPEC],
        out_shape=[jax.ShapeDtypeStruct((B, 3, N, C), _f32), _STAT_SHAPE],
    )(xfu, G, wxt, wdt, row(p['bd1']))
    a2, c2 = _fold2(st2, N * 3, p['gd1'], p['td1'])

    w2flat = jnp.concatenate([p['Wd2'][:, :, 0, t].T for t in range(3)], axis=0)
    z, st3 = pl.pallas_call(
        _k4,
        grid=(B,),
        in_specs=[_bspec((1, 3, N, C)), _vspec((1, C)), _vspec((1, C)),
                  _vspec((3 * C, C)), _vspec((1, C))],
        out_specs=[_bspec((1, N, C)), _STAT_SPEC],
        out_shape=[jax.ShapeDtypeStruct((B, N, C), _f32), _STAT_SHAPE],
    )(h, a2, c2, w2flat, row(p['bd2']))
    a3, c3 = _fold2(st3, N, p['gd2'], p['td2'])

    w2t = p['W2'][:, :, 0, 0].T
    od, p4, p4x, st4 = pl.pallas_call(
        _k5,
        grid=(B,),
        in_specs=[_bspec((1, N, C)), _vspec((1, C)), _vspec((1, C)),
                  _vspec((C, C)), _vspec((1, C))],
        out_specs=[_bspec((1, N, C)), _bspec((1, N, C)), _bspec((1, 8, C)),
                   _STAT_SPEC],
        out_shape=[jax.ShapeDtypeStruct((B, N, C), _f32),
                   jax.ShapeDtypeStruct((B, N, C), _f32),
                   jax.ShapeDtypeStruct((B, 8, C), _f32), _STAT_SHAPE],
    )(z, a3, c3, w2t, row(p['b2']))
    a4, c4 = _fold2(st4, N + 1, p['g_bn1'], p['t_bn1'])

    P0 = np.zeros((8, C, 8), np.float32)
    P1 = np.zeros((8, C, 8), np.float32)
    for g in range(8):
        for i in range(8):
            P0[g, 16 * g + i, i] = 1.0
            P1[g, 16 * g + 8 + i, i] = 1.0
    P0 = jnp.asarray(P0)
    P1 = jnp.asarray(P1)
    W3t = p['W3'][:, :, 0, 0].T
    mbd0 = jnp.einsum('gai,gbj,ij->ab', P0, P0, W3t)
    mbd1 = jnp.einsum('gai,gbj,ij->ab', P1, P1, W3t)
    b30 = jnp.einsum('gaj,j->a', P0, p['b3']).reshape(1, C)
    b31 = jnp.einsum('gaj,j->a', P1, p['b3']).reshape(1, C)
    ggl = jnp.einsum('gaj,j->a', P1, p['g_gn']).reshape(1, C)
    tgl = jnp.einsum('gaj,j->a', P1, p['t_gn']).reshape(1, C)
    m0 = jnp.asarray(
        np.where((np.arange(C) % 16) < 8, 1.0, 0.0).astype(np.float32)
    ).reshape(1, C)
    cold = np.arange(C)
    cnew = (cold % 64) * 2 + cold // 64
    wca = p['Wc1a'][:, :, 0, 0].T[jnp.asarray(cnew), :]

    p5, st5 = pl.pallas_call(
        _k6,
        grid=(B,),
        in_specs=[_bspec((1, N, C)), _bspec((1, N, C)), _bspec((1, 8, C)),
                  _vspec((1, C)), _vspec((1, C)), _vspec((1, C)), _vspec((1, C)),
                  _vspec((1, C)), _vspec((1, C)), _vspec((C, C)), _vspec((1, C)),
                  _vspec((1, C)), _vspec((C, C)), _vspec((C, C)), _vspec((1, C)),
                  _vspec((1, C)), _vspec((1, C)), _vspec((1, C)), _vspec((C, C)),
                  _vspec((1, C))],
        out_specs=[_bspec((1, N, C)), _STAT_SPEC],
        out_shape=[jax.ShapeDtypeStruct((B, N, C), _f32), _STAT_SHAPE],
    )(od, p4, p4x, a4, c4,
      row(p['cweight1'][0, :, 0, 0]), row(p['cbias1'][0, :, 0, 0]),
      row(p['sweight2'][0, :, 0, 0]), row(p['sbias2'][0, :, 0, 0]),
      w2t, row(p['b2']), m0, mbd0, mbd1, b30, b31, ggl, tgl,
      wca, row(p['bc1a']))
    a5, c5 = _fold2(st5, N, p['gc1a'], p['tc1a'])

    p6, st6 = pl.pallas_call(
        _k7,
        grid=(B,),
        in_specs=[_bspec((1, N, C)), _vspec((1, C)), _vspec((1, C)),
                  _vspec((C, C)), _vspec((1, C))],
        out_specs=[_bspec((1, N, C)), _STAT_SPEC],
        out_shape=[jax.ShapeDtypeStruct((B, N, C), _f32), _STAT_SHAPE],
    )(p5, a5, c5, p['Wc1b'][:, :, 0, 0].T, row(p['bc1b']))
    a6, c6 = _fold2(st6, N, p['gc1b'], p['tc1b'])

    bw_arr = jnp.full((1, 8, C), p['bw'][0], _f32)
    ex, p7, st7 = pl.pallas_call(
        _k8,
        grid=(B,),
        in_specs=[_bspec((1, N, C)), _vspec((1, C)), _vspec((1, C)),
                  _bspec((1, N, C)), _vspec((C, 1)), _vspec((1, 8, C)),
                  _vspec((C, C)), _vspec((1, C))],
        out_specs=[_bspec((1, N, C)), _bspec((1, N, C)), _STAT_SPEC],
        out_shape=[jax.ShapeDtypeStruct((B, N, C), _f32),
                   jax.ShapeDtypeStruct((B, N, C), _f32), _STAT_SHAPE],
    )(p6, a6, c6, od, p['Ww'][0, :, 0, 0].reshape(C, 1), bw_arr,
      p['Wg'][:, :, 0, 0].T, row(p['bg']))
    a7, c7 = _fold2(st7, N, p['gg'], p['tg'])

    bf_arr = jnp.full((1, 8, C), p['bf'][0], _f32)
    lg = pl.pallas_call(
        _k9,
        grid=(B,),
        in_specs=[_bspec((1, N, C)), _vspec((1, C)), _vspec((1, C)),
                  _bspec((1, N, C)), _vspec((C, 1)), _vspec((1, 8, C))],
        out_specs=_bspec((1, N, 1)),
        out_shape=jax.ShapeDtypeStruct((B, N, 1), _f32),
    )(p7, a7, c7, ex, p['Wf'][0, :, 0, 0].reshape(C, 1), bf_arr)

    return lg[:, :, 0]


# speedup vs baseline: 9.2360x; 1.0031x over previous
import functools

import jax
import jax.numpy as jnp
import numpy as np
from jax import lax
from jax.experimental import pallas as pl
from jax.experimental.pallas import tpu as pltpu
from jax.experimental.pallas import tpu_sc as plsc

B, N, C = 8, 2000, 128
NP = 2048
RB = 512
EPS = 1e-5
NROW = B * 9 * N
NWORK = 32
CHUNK = 384
NCH = 384
BIG = np.int32(1 << 30)

_f32 = jnp.float32


def _k1(x_ref, w_ref, b_ref, y_ref):
    y_ref[0] = jnp.dot(x_ref[0], w_ref[...], preferred_element_type=_f32) + b_ref[0]


def _k2(y_ref, yT_ref, g_ref, m_ref, d_ref, t_ref, gc_ref, mc_ref, dc_ref,
        tc_ref, xf_ref, idx_ref):
    b = pl.program_id(0)
    xf = jax.nn.relu(g_ref[0] * (y_ref[0] - m_ref[0]) / d_ref[0] + t_ref[0])
    xf_ref[0] = xf
    xfT = jax.nn.relu(gc_ref[...] * (yT_ref[0] - mc_ref[...]) / dc_ref[...]
                      + tc_ref[...])
    xx = jnp.sum(xf * xf, axis=1, keepdims=True)
    xxrow = jnp.sum(xfT * xfT, axis=0, keepdims=True)
    cols = lax.broadcasted_iota(jnp.int32, (RB, NP), 1)
    off = (b * NP).astype(jnp.int32)
    for p in range(NP // RB):
        xfR = xf[p * RB:(p + 1) * RB, :]
        mm = jnp.dot(xfR, xfT, preferred_element_type=_f32)
        vals = (2.0 * mm - xx[p * RB:(p + 1) * RB]) - xxrow
        vals = jnp.where(cols < N, vals, -jnp.inf)
        picks = []
        for _ in range(9):
            mx = jnp.max(vals, axis=1, keepdims=True)
            j = jnp.min(jnp.where(vals == mx, cols, BIG), axis=1, keepdims=True)
            picks.append(j)
            vals = jnp.where(cols == j, -jnp.inf, vals)
        idx_ref[0, pl.ds(p * RB, RB), :] = jnp.concatenate(picks, axis=1) + off


def _sc_gather(table_hbm, idx_hbm, out_hbm, idx0, idx1, rows0, rows1,
               sg0, sg1, so0, so1):
    wid = lax.axis_index("s") * 2 + lax.axis_index("c")
    nch = NCH // NWORK
    base = wid * nch
    idxb = [idx0, idx1]
    rowsb = [rows0, rows1]
    sg = [sg0, sg1]
    so = [so0, so1]
    for r0 in range(nch):
        p = r0 % 2
        if r0 >= 2:
            pltpu.make_async_copy(rowsb[p], out_hbm.at[base + r0 - 2],
                                  so[p]).wait()
        pltpu.sync_copy(idx_hbm.at[base + r0], idxb[p])
        pltpu.async_copy(table_hbm.at[idxb[p]], rowsb[p], sg[p])
        if r0 >= 1:
            q = (r0 - 1) % 2
            pltpu.make_async_copy(table_hbm.at[idxb[q]], rowsb[q], sg[q]).wait()
            pltpu.async_copy(rowsb[q], out_hbm.at[base + r0 - 1], so[q])
    pl2 = (nch - 1) % 2
    pltpu.make_async_copy(table_hbm.at[idxb[pl2]], rowsb[pl2], sg[pl2]).wait()
    pltpu.async_copy(rowsb[pl2], out_hbm.at[base + nch - 1], so[pl2])
    pltpu.make_async_copy(rowsb[1 - pl2], out_hbm.at[base + nch - 2],
                          so[1 - pl2]).wait()
    pltpu.make_async_copy(rowsb[pl2], out_hbm.at[base + nch - 1],
                          so[pl2]).wait()


def _gather_sc(table, idx2d):
    mesh = plsc.VectorSubcoreMesh(core_axis_name="c", subcore_axis_name="s")
    fn = functools.partial(
        pl.kernel,
        mesh=mesh,
        out_type=jax.ShapeDtypeStruct((NCH, CHUNK, C), _f32),
        scratch_types=[
            pltpu.VMEM((CHUNK,), jnp.int32),
            pltpu.VMEM((CHUNK,), jnp.int32),
            pltpu.VMEM((CHUNK, C), _f32),
            pltpu.VMEM((CHUNK, C), _f32),
            pltpu.SemaphoreType.DMA,
            pltpu.SemaphoreType.DMA,
            pltpu.SemaphoreType.DMA,
            pltpu.SemaphoreType.DMA,
        ],
    )(_sc_gather)
    return fn(table, idx2d)


def _stats(st_ref, s0, ssd):
    st_ref[0] = jnp.concatenate([s0, ssd, jnp.zeros((6, C), _f32)], axis=0)


def _k3(xf_ref, G_ref, wxt_ref, wdt_ref, b_ref, h_ref, st_ref):
    xfb = xf_ref[0]
    term1 = b_ref[0]
    for t in range(3):
        term1 = term1 + jnp.dot(xfb, wxt_ref[t], preferred_element_type=_f32)
    s0 = jnp.zeros((1, C), _f32)
    for s in range(3):
        acc = term1
        for t in range(3):
            d = 2.0 * jnp.minimum(0.0, xfb - G_ref[0, 3 * s + t])
            acc = acc + jnp.dot(d, wdt_ref[t], preferred_element_type=_f32)
        h_ref[0, s] = acc
        s0 = s0 + jnp.sum(acc, axis=0, keepdims=True)
    mb = s0 * _f32(1.0 / (3 * N))
    ssd = jnp.zeros((1, C), _f32)
    for s in range(3):
        dv = h_ref[0, s] - mb
        ssd = ssd + jnp.sum(dv * dv, axis=0, keepdims=True)
    _stats(st_ref, s0, ssd)


def _k4(h_ref, a_ref, c_ref, w_ref, b_ref, z_ref, st_ref):
    hcat = jnp.concatenate(
        [jax.nn.relu(h_ref[0, t] * a_ref[0] + c_ref[0]) for t in range(3)],
        axis=1)
    z = jnp.dot(hcat, w_ref[...], preferred_element_type=_f32) + b_ref[0]
    z_ref[0] = z
    s0 = jnp.sum(z, axis=0, keepdims=True)
    dv = z - s0 * _f32(1.0 / N)
    _stats(st_ref, s0, jnp.sum(dv * dv, axis=0, keepdims=True))


def _k5(z_ref, a_ref, c_ref, w2_ref, b2_ref, od_ref, p4_ref, p4x_ref, st_ref):
    od = jax.nn.relu(z_ref[0] * a_ref[0] + c_ref[0])
    od_ref[0] = od
    xmean = jnp.sum(od, axis=0, keepdims=True) * _f32(1.0 / N)
    pre4 = jnp.dot(od, w2_ref[...], preferred_element_type=_f32) + b2_ref[0]
    pre4x = jnp.dot(xmean, w2_ref[...], preferred_element_type=_f32) + b2_ref[0]
    p4_ref[0] = pre4
    p4x_ref[0] = jnp.concatenate([pre4x, jnp.zeros((7, C), _f32)], axis=0)
    s0 = jnp.sum(pre4, axis=0, keepdims=True) + pre4x
    mb = s0 * _f32(1.0 / (N + 1))
    dv = pre4 - mb
    dx = pre4x - mb
    _stats(st_ref, s0,
           jnp.sum(dv * dv, axis=0, keepdims=True) + dx * dx)


def _k6(od_ref, p4_ref, p4x_ref, a_ref, c_ref, cw_ref, cb_ref, sw_ref, sb_ref,
        w2_ref, b2_ref, m0_ref, mbd0_ref, mbd1_ref, b30_ref, b31_ref, gg_ref,
        tg_ref, wca_ref, bca_ref, p5_ref, st_ref):
    b = pl.program_id(0)
    od = od_ref[0]
    yn = jax.nn.relu(p4_ref[0] * a_ref[0] + c_ref[0])
    ynx = jax.nn.relu(p4x_ref[0, 0:1, :] * a_ref[0] + c_ref[0])
    xh2 = cw_ref[0] * yn + cb_ref[0]
    xw2 = sw_ref[0] * ynx + sb_ref[0]
    att_h = jax.nn.sigmoid(
        jnp.dot(xh2, w2_ref[...], preferred_element_type=_f32) + b2_ref[0])
    att_w = jax.nn.sigmoid(
        jnp.dot(xw2, w2_ref[...], preferred_element_type=_f32) + b2_ref[0])
    out1 = od * att_h * att_w
    m0 = m0_ref[0]
    mean_all = jnp.sum(out1, axis=0, keepdims=True) * _f32(1.0 / N)
    logit0 = jnp.dot(mean_all * m0, mbd0_ref[...],
                     preferred_element_type=_f32) + b30_ref[0]
    sig0 = jax.nn.sigmoid(logit0)
    dv0 = out1 - mean_all
    va = jnp.sum(dv0 * dv0, axis=0, keepdims=True) * _f32(1.0 / N)
    xsn = dv0 / jnp.sqrt(va + EPS) * gg_ref[0] + tg_ref[0]
    logit1 = jnp.dot(xsn * (1.0 - m0), mbd1_ref[...],
                     preferred_element_type=_f32) + b31_ref[0]
    sig1 = jax.nn.sigmoid(logit1)
    comb = out1 * (m0 * sig0 + (1.0 - m0) * sig1)
    t1 = jnp.dot(comb, wca_ref[...], preferred_element_type=_f32) + bca_ref[0]
    mu1 = jnp.sum(t1, axis=0, keepdims=True) * _f32(1.0 / N)
    dv1 = t1 - mu1
    va1 = jnp.sum(dv1 * dv1, axis=0, keepdims=True) * _f32(1.0 / N)
    t1n = dv1 / jnp.sqrt(va1 + EPS)
    p5_ref[0] = t1n
    s0 = jnp.sum(t1n, axis=0, keepdims=True)
    dv = t1n - s0 * _f32(1.0 / N)
    _stats(st_ref, s0, jnp.sum(dv * dv, axis=0, keepdims=True))


def _k7(p5_ref, a_ref, c_ref, w_ref, b_ref, p6_ref, st_ref):
    h1 = jax.nn.relu(p5_ref[0] * a_ref[0] + c_ref[0])
    h2 = jnp.dot(h1, w_ref[...], preferred_element_type=_f32) + b_ref[0]
    mu = jnp.sum(h2, axis=0, keepdims=True) * _f32(1.0 / N)
    dv0 = h2 - mu
    va = jnp.sum(dv0 * dv0, axis=0, keepdims=True) * _f32(1.0 / N)
    h2n = dv0 / jnp.sqrt(va + EPS)
    p6_ref[0] = h2n
    s0 = jnp.sum(h2n, axis=0, keepdims=True)
    dv = h2n - s0 * _f32(1.0 / N)
    _stats(st_ref, s0, jnp.sum(dv * dv, axis=0, keepdims=True))


def _bf(v):
    return v.astype(jnp.bfloat16).astype(_f32)


def _k8(p6_ref, a_ref, c_ref, od_ref, ww_ref, bw_ref, wg_ref, bg_ref,
        ex_ref, p7_ref, st_ref):
    b = pl.program_id(0)
    t3 = p6_ref[0] * a_ref[0] + c_ref[0]
    ex = jax.nn.relu(t3 + od_ref[0])
    ex_ref[0] = ex
    wv = jnp.dot(ex, ww_ref[...], preferred_element_type=_f32) \
        + bw_ref[0, 0:1, 0:1]
    w2 = _bf(jax.nn.relu(jnp.tanh(wv)))
    s = jnp.sum(w2 * w2, axis=0, keepdims=True)[0:1, 0:1]
    dd = jnp.sqrt(1.0 / (_f32(N) * s + 1.0))
    db = _bf(dd)
    qo = _bf(_bf(db * _bf(s)) * db)
    qd = _bf(_bf(db * _bf(s + 1.0)) * db)
    exb = _bf(ex)
    colsum = jnp.sum(exb, axis=0, keepdims=True)
    u = qo * colsum + (qd - qo) * exb
    pre7 = jnp.dot(u, wg_ref[...], preferred_element_type=_f32) + bg_ref[0]
    p7_ref[0] = pre7
    s0 = jnp.sum(pre7, axis=0, keepdims=True)
    dv = pre7 - s0 * _f32(1.0 / N)
    _stats(st_ref, s0, jnp.sum(dv * dv, axis=0, keepdims=True))


def _k9(p7_ref, a_ref, c_ref, ex_ref, wf_ref, bf_ref, lg_ref):
    fin = jax.nn.relu(p7_ref[0] * a_ref[0] + c_ref[0]) + ex_ref[0]
    lg_ref[0] = jnp.dot(fin, wf_ref[...], preferred_element_type=_f32) \
        + bf_ref[0, 0:1, 0:1]


def _vspec(shape):
    nd = len(shape)
    return pl.BlockSpec(shape, lambda b, _nd=nd: (0,) * _nd)


def _bspec(shape):
    nd = len(shape)
    return pl.BlockSpec(shape, lambda b, _nd=nd: (b,) + (0,) * (_nd - 1))


_STAT_SPEC = pl.BlockSpec((1, 8, C), lambda b: (b, 0, 0))
_STAT_SHAPE = jax.ShapeDtypeStruct((B, 8, C), jnp.float32)


def _fold2(st, n_b, g, t):
    s0 = st[:, 0, :]
    ssd = st[:, 1, :]
    mb = s0 / n_b
    n = B * n_b
    m = jnp.sum(s0, axis=0) / n
    v = (jnp.sum(ssd, axis=0) + n_b * jnp.sum((mb - m) ** 2, axis=0)) / n
    a = g / jnp.sqrt(v + EPS)
    return a.reshape(1, C), (t - m * a).reshape(1, C)


def kernel(x, params):
    p = params
    xin = x[:, 0]
    xpad = jnp.concatenate(
        [xin, jnp.zeros((B, NP - N, 4), _f32)], axis=1)

    y = pl.pallas_call(
        _k1,
        grid=(B,),
        in_specs=[_bspec((1, NP, 4)), _vspec((4, C)), _vspec((1, C))],
        out_specs=_bspec((1, NP, C)),
        out_shape=jax.ShapeDtypeStruct((B, NP, C), _f32),
    )(xpad, p['Wci'][:, :, 0, 0].T, p['bci'].reshape(1, C))

    yv = y[:, :N, :]
    m1 = jnp.mean(yv, axis=(0, 1))
    v1 = jnp.var(yv, axis=(0, 1))
    d1 = jnp.sqrt(v1 + EPS)
    yT = jnp.transpose(y, (0, 2, 1))

    row = lambda a: a.reshape(1, C)
    col = lambda a: a.reshape(C, 1)

    xf, idx = pl.pallas_call(
        _k2,
        grid=(B,),
        in_specs=[_bspec((1, NP, C)), _bspec((1, C, NP)),
                  _vspec((1, C)), _vspec((1, C)), _vspec((1, C)), _vspec((1, C)),
                  _vspec((C, 1)), _vspec((C, 1)), _vspec((C, 1)), _vspec((C, 1))],
        out_specs=[_bspec((1, NP, C)), _bspec((1, NP, 9))],
        out_shape=[jax.ShapeDtypeStruct((B, NP, C), _f32),
                   jax.ShapeDtypeStruct((B, NP, 9), jnp.int32)],
    )(y, yT, row(p['gci']), row(m1), row(d1), row(p['tci']),
      col(p['gci']), col(m1), col(d1), col(p['tci']))

    idx_t = jnp.transpose(idx[:, :N, :], (0, 2, 1)).reshape(-1)
    idx2d = jnp.concatenate(
        [idx_t, jnp.zeros((NCH * CHUNK - NROW,), jnp.int32)]).reshape(NCH, CHUNK)
    table = xf.reshape(B * NP, C)
    rows = _gather_sc(table, idx2d)
    G = rows.reshape(-1, C)[:NROW].reshape(B, 9, N, C)

    xfu = xf[:, :N, :]

    Wd1 = p['Wd1']
    wxt = jnp.stack([Wd1[:, :C, 0, t].T for t in range(3)], axis=0)
    wdt = jnp.stack([Wd1[:, C:, 0, t].T for t in range(3)], axis=0)
    h, st2 = pl.pallas_call(
        _k3,
        grid=(B,),
        in_specs=[_bspec((1, N, C)), _bspec((1, 9, N, C)),
                  _vspec((3, C, C)), _vspec((3, C, C)), _vspec((1, C))],
        out_specs=[_bspec((1, 3, N, C)), _STAT_SPEC],
        out_shape=[jax.ShapeDtypeStruct((B, 3, N, C), _f32), _STAT_SHAPE],
    )(xfu, G, wxt, wdt, row(p['bd1']))
    a2, c2 = _fold2(st2, N * 3, p['gd1'], p['td1'])

    w2flat = jnp.concatenate([p['Wd2'][:, :, 0, t].T for t in range(3)], axis=0)
    z, st3 = pl.pallas_call(
        _k4,
        grid=(B,),
        in_specs=[_bspec((1, 3, N, C)), _vspec((1, C)), _vspec((1, C)),
                  _vspec((3 * C, C)), _vspec((1, C))],
        out_specs=[_bspec((1, N, C)), _STAT_SPEC],
        out_shape=[jax.ShapeDtypeStruct((B, N, C), _f32), _STAT_SHAPE],
    )(h, a2, c2, w2flat, row(p['bd2']))
    a3, c3 = _fold2(st3, N, p['gd2'], p['td2'])

    w2t = p['W2'][:, :, 0, 0].T
    od, p4, p4x, st4 = pl.pallas_call(
        _k5,
        grid=(B,),
        in_specs=[_bspec((1, N, C)), _vspec((1, C)), _vspec((1, C)),
                  _vspec((C, C)), _vspec((1, C))],
        out_specs=[_bspec((1, N, C)), _bspec((1, N, C)), _bspec((1, 8, C)),
                   _STAT_SPEC],
        out_shape=[jax.ShapeDtypeStruct((B, N, C), _f32),
                   jax.ShapeDtypeStruct((B, N, C), _f32),
                   jax.ShapeDtypeStruct((B, 8, C), _f32), _STAT_SHAPE],
    )(z, a3, c3, w2t, row(p['b2']))
    a4, c4 = _fold2(st4, N + 1, p['g_bn1'], p['t_bn1'])

    P0 = np.zeros((8, C, 8), np.float32)
    P1 = np.zeros((8, C, 8), np.float32)
    for g in range(8):
        for i in range(8):
            P0[g, 16 * g + i, i] = 1.0
            P1[g, 16 * g + 8 + i, i] = 1.0
    P0 = jnp.asarray(P0)
    P1 = jnp.asarray(P1)
    W3t = p['W3'][:, :, 0, 0].T
    mbd0 = jnp.einsum('gai,gbj,ij->ab', P0, P0, W3t)
    mbd1 = jnp.einsum('gai,gbj,ij->ab', P1, P1, W3t)
    b30 = jnp.einsum('gaj,j->a', P0, p['b3']).reshape(1, C)
    b31 = jnp.einsum('gaj,j->a', P1, p['b3']).reshape(1, C)
    ggl = jnp.einsum('gaj,j->a', P1, p['g_gn']).reshape(1, C)
    tgl = jnp.einsum('gaj,j->a', P1, p['t_gn']).reshape(1, C)
    m0 = jnp.asarray(
        np.where((np.arange(C) % 16) < 8, 1.0, 0.0).astype(np.float32)
    ).reshape(1, C)
    cold = np.arange(C)
    cnew = (cold % 64) * 2 + cold // 64
    wca = p['Wc1a'][:, :, 0, 0].T[jnp.asarray(cnew), :]

    p5, st5 = pl.pallas_call(
        _k6,
        grid=(B,),
        in_specs=[_bspec((1, N, C)), _bspec((1, N, C)), _bspec((1, 8, C)),
                  _vspec((1, C)), _vspec((1, C)), _vspec((1, C)), _vspec((1, C)),
                  _vspec((1, C)), _vspec((1, C)), _vspec((C, C)), _vspec((1, C)),
                  _vspec((1, C)), _vspec((C, C)), _vspec((C, C)), _vspec((1, C)),
                  _vspec((1, C)), _vspec((1, C)), _vspec((1, C)), _vspec((C, C)),
                  _vspec((1, C))],
        out_specs=[_bspec((1, N, C)), _STAT_SPEC],
        out_shape=[jax.ShapeDtypeStruct((B, N, C), _f32), _STAT_SHAPE],
    )(od, p4, p4x, a4, c4,
      row(p['cweight1'][0, :, 0, 0]), row(p['cbias1'][0, :, 0, 0]),
      row(p['sweight2'][0, :, 0, 0]), row(p['sbias2'][0, :, 0, 0]),
      w2t, row(p['b2']), m0, mbd0, mbd1, b30, b31, ggl, tgl,
      wca, row(p['bc1a']))
    a5, c5 = _fold2(st5, N, p['gc1a'], p['tc1a'])

    p6, st6 = pl.pallas_call(
        _k7,
        grid=(B,),
        in_specs=[_bspec((1, N, C)), _vspec((1, C)), _vspec((1, C)),
                  _vspec((C, C)), _vspec((1, C))],
        out_specs=[_bspec((1, N, C)), _STAT_SPEC],
        out_shape=[jax.ShapeDtypeStruct((B, N, C), _f32), _STAT_SHAPE],
    )(p5, a5, c5, p['Wc1b'][:, :, 0, 0].T, row(p['bc1b']))
    a6, c6 = _fold2(st6, N, p['gc1b'], p['tc1b'])

    bw_arr = jnp.full((1, 8, C), p['bw'][0], _f32)
    ex, p7, st7 = pl.pallas_call(
        _k8,
        grid=(B,),
        in_specs=[_bspec((1, N, C)), _vspec((1, C)), _vspec((1, C)),
                  _bspec((1, N, C)), _vspec((C, 1)), _vspec((1, 8, C)),
                  _vspec((C, C)), _vspec((1, C))],
        out_specs=[_bspec((1, N, C)), _bspec((1, N, C)), _STAT_SPEC],
        out_shape=[jax.ShapeDtypeStruct((B, N, C), _f32),
                   jax.ShapeDtypeStruct((B, N, C), _f32), _STAT_SHAPE],
    )(p6, a6, c6, od, p['Ww'][0, :, 0, 0].reshape(C, 1), bw_arr,
      p['Wg'][:, :, 0, 0].T, row(p['bg']))
    a7, c7 = _fold2(st7, N, p['gg'], p['tg'])

    bf_arr = jnp.full((1, 8, C), p['bf'][0], _f32)
    lg = pl.pallas_call(
        _k9,
        grid=(B,),
        in_specs=[_bspec((1, N, C)), _vspec((1, C)), _vspec((1, C)),
                  _bspec((1, N, C)), _vspec((C, 1)), _vspec((1, 8, C))],
        out_specs=_bspec((1, N, 1)),
        out_shape=jax.ShapeDtypeStruct((B, N, 1), _f32),
    )(p7, a7, c7, ex, p['Wf'][0, :, 0, 0].reshape(C, 1), bf_arr)

    return lg[:, :, 0]


# speedup vs baseline: 12.5618x; 1.3601x over previous
import functools

import jax
import jax.numpy as jnp
import numpy as np
from jax import lax
from jax.experimental import pallas as pl
from jax.experimental.pallas import tpu as pltpu
from jax.experimental.pallas import tpu_sc as plsc

B, N, C = 8, 2000, 128
NP = 2048
RB = 512
EPS = 1e-5
NROW = B * 9 * N
NWORK = 32
CHUNK = 400
NCH = 360
LAPS = 12
BIG = np.int32(1 << 30)

_f32 = jnp.float32


def _k1(x_ref, w_ref, b_ref, y_ref):
    y_ref[0] = jnp.dot(x_ref[0], w_ref[...], preferred_element_type=_f32) + b_ref[0]


def _k2(y_ref, yT_ref, g_ref, m_ref, d_ref, t_ref, gc_ref, mc_ref, dc_ref,
        tc_ref, xf_ref, idx_ref):
    b = pl.program_id(0)
    xf = jax.nn.relu(g_ref[0] * (y_ref[0] - m_ref[0]) / d_ref[0] + t_ref[0])
    xf_ref[0] = xf
    xfT = jax.nn.relu(gc_ref[...] * (yT_ref[0] - mc_ref[...]) / dc_ref[...]
                      + tc_ref[...])
    xx = jnp.sum(xf * xf, axis=1, keepdims=True)
    xxrow = jnp.sum(xfT * xfT, axis=0, keepdims=True)
    cols = lax.broadcasted_iota(jnp.int32, (RB, NP), 1)
    off = (b * NP).astype(jnp.int32)
    for p in range(NP // RB):
        xfR = xf[p * RB:(p + 1) * RB, :]
        mm = jnp.dot(xfR, xfT, preferred_element_type=_f32)
        vals = (2.0 * mm - xx[p * RB:(p + 1) * RB]) - xxrow
        vals = jnp.where(cols < N, vals, -jnp.inf)
        picks = []
        for _ in range(9):
            mx = jnp.max(vals, axis=1, keepdims=True)
            j = jnp.min(jnp.where(vals == mx, cols, BIG), axis=1, keepdims=True)
            picks.append(j)
            vals = jnp.where(cols == j, -jnp.inf, vals)
        idx_ref[0, pl.ds(p * RB, RB), :] = jnp.concatenate(picks, axis=1) + off


def _sc_gather(table_hbm, idx_hbm, out_hbm, idx0, idx1, rows0, rows1,
               sg0, sg1, so0, so1):
    wid = lax.axis_index("s") * 2 + lax.axis_index("c")
    idxb = [idx0, idx1]
    rowsb = [rows0, rows1]
    sg = [sg0, sg1]
    so = [so0, so1]
    for i in range(LAPS):
        p = i % 2
        cur = wid + NWORK * i
        if i >= 2:
            pltpu.make_async_copy(rowsb[p], out_hbm.at[wid + NWORK * (i - 2)],
                                  so[p]).wait()

        @pl.when(cur < NCH)
        def _(p=p, cur=cur):
            pltpu.sync_copy(idx_hbm.at[cur], idxb[p])
            pltpu.async_copy(table_hbm.at[idxb[p]], rowsb[p], sg[p])

        if i >= 1:
            q = (i - 1) % 2
            prev = wid + NWORK * (i - 1)

            @pl.when(prev < NCH)
            def _(q=q, prev=prev):
                pltpu.make_async_copy(table_hbm.at[idxb[q]], rowsb[q],
                                      sg[q]).wait()
                pltpu.async_copy(rowsb[q], out_hbm.at[prev], so[q])

    last = wid + NWORK * (LAPS - 1)
    pl2 = (LAPS - 1) % 2

    @pl.when(last < NCH)
    def _():
        pltpu.make_async_copy(table_hbm.at[idxb[pl2]], rowsb[pl2],
                              sg[pl2]).wait()
        pltpu.async_copy(rowsb[pl2], out_hbm.at[last], so[pl2])
        pltpu.make_async_copy(rowsb[pl2], out_hbm.at[last], so[pl2]).wait()

    pltpu.make_async_copy(rowsb[1 - pl2], out_hbm.at[wid + NWORK * (LAPS - 2)],
                          so[1 - pl2]).wait()


def _gather_sc(table, idx2d):
    mesh = plsc.VectorSubcoreMesh(core_axis_name="c", subcore_axis_name="s")
    fn = functools.partial(
        pl.kernel,
        mesh=mesh,
        out_type=jax.ShapeDtypeStruct((NCH, CHUNK, C), _f32),
        scratch_types=[
            pltpu.VMEM((CHUNK,), jnp.int32),
            pltpu.VMEM((CHUNK,), jnp.int32),
            pltpu.VMEM((CHUNK, C), _f32),
            pltpu.VMEM((CHUNK, C), _f32),
            pltpu.SemaphoreType.DMA,
            pltpu.SemaphoreType.DMA,
            pltpu.SemaphoreType.DMA,
            pltpu.SemaphoreType.DMA,
        ],
    )(_sc_gather)
    return fn(table, idx2d)


def _stats(st_ref, s0, ssd):
    st_ref[0] = jnp.concatenate([s0, ssd, jnp.zeros((6, C), _f32)], axis=0)


def _k3(xf_ref, G_ref, wxt_ref, wdt_ref, b_ref, h_ref, st_ref):
    xfb = xf_ref[0]
    term1 = b_ref[0]
    for t in range(3):
        term1 = term1 + jnp.dot(xfb, wxt_ref[t], preferred_element_type=_f32)
    s0 = jnp.zeros((1, C), _f32)
    for s in range(3):
        acc = term1
        for t in range(3):
            d = 2.0 * jnp.minimum(0.0, xfb - G_ref[0, 3 * s + t])
            acc = acc + jnp.dot(d, wdt_ref[t], preferred_element_type=_f32)
        h_ref[0, s] = acc
        s0 = s0 + jnp.sum(acc, axis=0, keepdims=True)
    mb = s0 * _f32(1.0 / (3 * N))
    ssd = jnp.zeros((1, C), _f32)
    for s in range(3):
        dv = h_ref[0, s] - mb
        ssd = ssd + jnp.sum(dv * dv, axis=0, keepdims=True)
    _stats(st_ref, s0, ssd)


def _k4(h_ref, a_ref, c_ref, w_ref, b_ref, z_ref, st_ref):
    hcat = jnp.concatenate(
        [jax.nn.relu(h_ref[0, t] * a_ref[0] + c_ref[0]) for t in range(3)],
        axis=1)
    z = jnp.dot(hcat, w_ref[...], preferred_element_type=_f32) + b_ref[0]
    z_ref[0] = z
    s0 = jnp.sum(z, axis=0, keepdims=True)
    dv = z - s0 * _f32(1.0 / N)
    _stats(st_ref, s0, jnp.sum(dv * dv, axis=0, keepdims=True))


def _k5(z_ref, a_ref, c_ref, w2_ref, b2_ref, od_ref, p4_ref, p4x_ref, st_ref):
    od = jax.nn.relu(z_ref[0] * a_ref[0] + c_ref[0])
    od_ref[0] = od
    xmean = jnp.sum(od, axis=0, keepdims=True) * _f32(1.0 / N)
    pre4 = jnp.dot(od, w2_ref[...], preferred_element_type=_f32) + b2_ref[0]
    pre4x = jnp.dot(xmean, w2_ref[...], preferred_element_type=_f32) + b2_ref[0]
    p4_ref[0] = pre4
    p4x_ref[0] = jnp.concatenate([pre4x, jnp.zeros((7, C), _f32)], axis=0)
    s0 = jnp.sum(pre4, axis=0, keepdims=True) + pre4x
    mb = s0 * _f32(1.0 / (N + 1))
    dv = pre4 - mb
    dx = pre4x - mb
    _stats(st_ref, s0,
           jnp.sum(dv * dv, axis=0, keepdims=True) + dx * dx)


def _k6(od_ref, p4_ref, p4x_ref, a_ref, c_ref, cw_ref, cb_ref, sw_ref, sb_ref,
        w2_ref, b2_ref, m0_ref, mbd0_ref, mbd1_ref, b30_ref, b31_ref, gg_ref,
        tg_ref, wca_ref, bca_ref, p5_ref, st_ref):
    b = pl.program_id(0)
    od = od_ref[0]
    yn = jax.nn.relu(p4_ref[0] * a_ref[0] + c_ref[0])
    ynx = jax.nn.relu(p4x_ref[0, 0:1, :] * a_ref[0] + c_ref[0])
    xh2 = cw_ref[0] * yn + cb_ref[0]
    xw2 = sw_ref[0] * ynx + sb_ref[0]
    att_h = jax.nn.sigmoid(
        jnp.dot(xh2, w2_ref[...], preferred_element_type=_f32) + b2_ref[0])
    att_w = jax.nn.sigmoid(
        jnp.dot(xw2, w2_ref[...], preferred_element_type=_f32) + b2_ref[0])
    out1 = od * att_h * att_w
    m0 = m0_ref[0]
    mean_all = jnp.sum(out1, axis=0, keepdims=True) * _f32(1.0 / N)
    logit0 = jnp.dot(mean_all * m0, mbd0_ref[...],
                     preferred_element_type=_f32) + b30_ref[0]
    sig0 = jax.nn.sigmoid(logit0)
    dv0 = out1 - mean_all
    va = jnp.sum(dv0 * dv0, axis=0, keepdims=True) * _f32(1.0 / N)
    xsn = dv0 / jnp.sqrt(va + EPS) * gg_ref[0] + tg_ref[0]
    logit1 = jnp.dot(xsn * (1.0 - m0), mbd1_ref[...],
                     preferred_element_type=_f32) + b31_ref[0]
    sig1 = jax.nn.sigmoid(logit1)
    comb = out1 * (m0 * sig0 + (1.0 - m0) * sig1)
    t1 = jnp.dot(comb, wca_ref[...], preferred_element_type=_f32) + bca_ref[0]
    mu1 = jnp.sum(t1, axis=0, keepdims=True) * _f32(1.0 / N)
    dv1 = t1 - mu1
    va1 = jnp.sum(dv1 * dv1, axis=0, keepdims=True) * _f32(1.0 / N)
    t1n = dv1 / jnp.sqrt(va1 + EPS)
    p5_ref[0] = t1n
    s0 = jnp.sum(t1n, axis=0, keepdims=True)
    dv = t1n - s0 * _f32(1.0 / N)
    _stats(st_ref, s0, jnp.sum(dv * dv, axis=0, keepdims=True))


def _k7(p5_ref, a_ref, c_ref, w_ref, b_ref, p6_ref, st_ref):
    h1 = jax.nn.relu(p5_ref[0] * a_ref[0] + c_ref[0])
    h2 = jnp.dot(h1, w_ref[...], preferred_element_type=_f32) + b_ref[0]
    mu = jnp.sum(h2, axis=0, keepdims=True) * _f32(1.0 / N)
    dv0 = h2 - mu
    va = jnp.sum(dv0 * dv0, axis=0, keepdims=True) * _f32(1.0 / N)
    h2n = dv0 / jnp.sqrt(va + EPS)
    p6_ref[0] = h2n
    s0 = jnp.sum(h2n, axis=0, keepdims=True)
    dv = h2n - s0 * _f32(1.0 / N)
    _stats(st_ref, s0, jnp.sum(dv * dv, axis=0, keepdims=True))


def _bf(v):
    return v.astype(jnp.bfloat16).astype(_f32)


def _k8(p6_ref, a_ref, c_ref, od_ref, ww_ref, bw_ref, wg_ref, bg_ref,
        ex_ref, p7_ref, st_ref):
    b = pl.program_id(0)
    t3 = p6_ref[0] * a_ref[0] + c_ref[0]
    ex = jax.nn.relu(t3 + od_ref[0])
    ex_ref[0] = ex
    wv = jnp.dot(ex, ww_ref[...], preferred_element_type=_f32) \
        + bw_ref[0, 0:1, 0:1]
    w2 = _bf(jax.nn.relu(jnp.tanh(wv)))
    s = jnp.sum(w2 * w2, axis=0, keepdims=True)[0:1, 0:1]
    dd = jnp.sqrt(1.0 / (_f32(N) * s + 1.0))
    db = _bf(dd)
    qo = _bf(_bf(db * _bf(s)) * db)
    qd = _bf(_bf(db * _bf(s + 1.0)) * db)
    exb = _bf(ex)
    colsum = jnp.sum(exb, axis=0, keepdims=True)
    u = qo * colsum + (qd - qo) * exb
    pre7 = jnp.dot(u, wg_ref[...], preferred_element_type=_f32) + bg_ref[0]
    p7_ref[0] = pre7
    s0 = jnp.sum(pre7, axis=0, keepdims=True)
    dv = pre7 - s0 * _f32(1.0 / N)
    _stats(st_ref, s0, jnp.sum(dv * dv, axis=0, keepdims=True))


def _k9(p7_ref, a_ref, c_ref, ex_ref, wf_ref, bf_ref, lg_ref):
    fin = jax.nn.relu(p7_ref[0] * a_ref[0] + c_ref[0]) + ex_ref[0]
    lg_ref[0] = jnp.dot(fin, wf_ref[...], preferred_element_type=_f32) \
        + bf_ref[0, 0:1, 0:1]


def _vspec(shape):
    nd = len(shape)
    return pl.BlockSpec(shape, lambda b, _nd=nd: (0,) * _nd)


def _bspec(shape):
    nd = len(shape)
    return pl.BlockSpec(shape, lambda b, _nd=nd: (b,) + (0,) * (_nd - 1))


_STAT_SPEC = pl.BlockSpec((1, 8, C), lambda b: (b, 0, 0))
_STAT_SHAPE = jax.ShapeDtypeStruct((B, 8, C), jnp.float32)


def _fold2(st, n_b, g, t):
    s0 = st[:, 0, :]
    ssd = st[:, 1, :]
    mb = s0 / n_b
    n = B * n_b
    m = jnp.sum(s0, axis=0) / n
    v = (jnp.sum(ssd, axis=0) + n_b * jnp.sum((mb - m) ** 2, axis=0)) / n
    a = g / jnp.sqrt(v + EPS)
    return a.reshape(1, C), (t - m * a).reshape(1, C)


def kernel(x, params):
    p = params
    xin = x[:, 0]
    xpad = jnp.concatenate(
        [xin, jnp.zeros((B, NP - N, 4), _f32)], axis=1)

    y = pl.pallas_call(
        _k1,
        grid=(B,),
        in_specs=[_bspec((1, NP, 4)), _vspec((4, C)), _vspec((1, C))],
        out_specs=_bspec((1, NP, C)),
        out_shape=jax.ShapeDtypeStruct((B, NP, C), _f32),
    )(xpad, p['Wci'][:, :, 0, 0].T, p['bci'].reshape(1, C))

    yv = y[:, :N, :]
    m1 = jnp.mean(yv, axis=(0, 1))
    v1 = jnp.var(yv, axis=(0, 1))
    d1 = jnp.sqrt(v1 + EPS)
    yT = jnp.transpose(y, (0, 2, 1))

    row = lambda a: a.reshape(1, C)
    col = lambda a: a.reshape(C, 1)

    xf, idx = pl.pallas_call(
        _k2,
        grid=(B,),
        in_specs=[_bspec((1, NP, C)), _bspec((1, C, NP)),
                  _vspec((1, C)), _vspec((1, C)), _vspec((1, C)), _vspec((1, C)),
                  _vspec((C, 1)), _vspec((C, 1)), _vspec((C, 1)), _vspec((C, 1))],
        out_specs=[_bspec((1, NP, C)), _bspec((1, NP, 9))],
        out_shape=[jax.ShapeDtypeStruct((B, NP, C), _f32),
                   jax.ShapeDtypeStruct((B, NP, 9), jnp.int32)],
    )(y, yT, row(p['gci']), row(m1), row(d1), row(p['tci']),
      col(p['gci']), col(m1), col(d1), col(p['tci']))

    idx2d = jnp.transpose(idx[:, :N, :], (0, 2, 1)).reshape(NCH, CHUNK)
    table = xf.reshape(B * NP, C)
    rows = _gather_sc(table, idx2d)
    G = rows.reshape(B, 9, N, C)

    xfu = xf[:, :N, :]

    Wd1 = p['Wd1']
    wxt = jnp.stack([Wd1[:, :C, 0, t].T for t in range(3)], axis=0)
    wdt = jnp.stack([Wd1[:, C:, 0, t].T for t in range(3)], axis=0)
    h, st2 = pl.pallas_call(
        _k3,
        grid=(B,),
        in_specs=[_bspec((1, N, C)), _bspec((1, 9, N, C)),
                  _vspec((3, C, C)), _vspec((3, C, C)), _vspec((1, C))],
        out_specs=[_bspec((1, 3, N, C)), _STAT_SPEC],
        out_shape=[jax.ShapeDtypeStruct((B, 3, N, C), _f32), _STAT_SHAPE],
    )(xfu, G, wxt, wdt, row(p['bd1']))
    a2, c2 = _fold2(st2, N * 3, p['gd1'], p['td1'])

    w2flat = jnp.concatenate([p['Wd2'][:, :, 0, t].T for t in range(3)], axis=0)
    z, st3 = pl.pallas_call(
        _k4,
        grid=(B,),
        in_specs=[_bspec((1, 3, N, C)), _vspec((1, C)), _vspec((1, C)),
                  _vspec((3 * C, C)), _vspec((1, C))],
        out_specs=[_bspec((1, N, C)), _STAT_SPEC],
        out_shape=[jax.ShapeDtypeStruct((B, N, C), _f32), _STAT_SHAPE],
    )(h, a2, c2, w2flat, row(p['bd2']))
    a3, c3 = _fold2(st3, N, p['gd2'], p['td2'])

    w2t = p['W2'][:, :, 0, 0].T
    od, p4, p4x, st4 = pl.pallas_call(
        _k5,
        grid=(B,),
        in_specs=[_bspec((1, N, C)), _vspec((1, C)), _vspec((1, C)),
                  _vspec((C, C)), _vspec((1, C))],
        out_specs=[_bspec((1, N, C)), _bspec((1, N, C)), _bspec((1, 8, C)),
                   _STAT_SPEC],
        out_shape=[jax.ShapeDtypeStruct((B, N, C), _f32),
                   jax.ShapeDtypeStruct((B, N, C), _f32),
                   jax.ShapeDtypeStruct((B, 8, C), _f32), _STAT_SHAPE],
    )(z, a3, c3, w2t, row(p['b2']))
    a4, c4 = _fold2(st4, N + 1, p['g_bn1'], p['t_bn1'])

    P0 = np.zeros((8, C, 8), np.float32)
    P1 = np.zeros((8, C, 8), np.float32)
    for g in range(8):
        for i in range(8):
            P0[g, 16 * g + i, i] = 1.0
            P1[g, 16 * g + 8 + i, i] = 1.0
    P0 = jnp.asarray(P0)
    P1 = jnp.asarray(P1)
    W3t = p['W3'][:, :, 0, 0].T
    mbd0 = jnp.einsum('gai,gbj,ij->ab', P0, P0, W3t)
    mbd1 = jnp.einsum('gai,gbj,ij->ab', P1, P1, W3t)
    b30 = jnp.einsum('gaj,j->a', P0, p['b3']).reshape(1, C)
    b31 = jnp.einsum('gaj,j->a', P1, p['b3']).reshape(1, C)
    ggl = jnp.einsum('gaj,j->a', P1, p['g_gn']).reshape(1, C)
    tgl = jnp.einsum('gaj,j->a', P1, p['t_gn']).reshape(1, C)
    m0 = jnp.asarray(
        np.where((np.arange(C) % 16) < 8, 1.0, 0.0).astype(np.float32)
    ).reshape(1, C)
    cold = np.arange(C)
    cnew = (cold % 64) * 2 + cold // 64
    wca = p['Wc1a'][:, :, 0, 0].T[jnp.asarray(cnew), :]

    p5, st5 = pl.pallas_call(
        _k6,
        grid=(B,),
        in_specs=[_bspec((1, N, C)), _bspec((1, N, C)), _bspec((1, 8, C)),
                  _vspec((1, C)), _vspec((1, C)), _vspec((1, C)), _vspec((1, C)),
                  _vspec((1, C)), _vspec((1, C)), _vspec((C, C)), _vspec((1, C)),
                  _vspec((1, C)), _vspec((C, C)), _vspec((C, C)), _vspec((1, C)),
                  _vspec((1, C)), _vspec((1, C)), _vspec((1, C)), _vspec((C, C)),
                  _vspec((1, C))],
        out_specs=[_bspec((1, N, C)), _STAT_SPEC],
        out_shape=[jax.ShapeDtypeStruct((B, N, C), _f32), _STAT_SHAPE],
    )(od, p4, p4x, a4, c4,
      row(p['cweight1'][0, :, 0, 0]), row(p['cbias1'][0, :, 0, 0]),
      row(p['sweight2'][0, :, 0, 0]), row(p['sbias2'][0, :, 0, 0]),
      w2t, row(p['b2']), m0, mbd0, mbd1, b30, b31, ggl, tgl,
      wca, row(p['bc1a']))
    a5, c5 = _fold2(st5, N, p['gc1a'], p['tc1a'])

    p6, st6 = pl.pallas_call(
        _k7,
        grid=(B,),
        in_specs=[_bspec((1, N, C)), _vspec((1, C)), _vspec((1, C)),
                  _vspec((C, C)), _vspec((1, C))],
        out_specs=[_bspec((1, N, C)), _STAT_SPEC],
        out_shape=[jax.ShapeDtypeStruct((B, N, C), _f32), _STAT_SHAPE],
    )(p5, a5, c5, p['Wc1b'][:, :, 0, 0].T, row(p['bc1b']))
    a6, c6 = _fold2(st6, N, p['gc1b'], p['tc1b'])

    bw_arr = jnp.full((1, 8, C), p['bw'][0], _f32)
    ex, p7, st7 = pl.pallas_call(
        _k8,
        grid=(B,),
        in_specs=[_bspec((1, N, C)), _vspec((1, C)), _vspec((1, C)),
                  _bspec((1, N, C)), _vspec((C, 1)), _vspec((1, 8, C)),
                  _vspec((C, C)), _vspec((1, C))],
        out_specs=[_bspec((1, N, C)), _bspec((1, N, C)), _STAT_SPEC],
        out_shape=[jax.ShapeDtypeStruct((B, N, C), _f32),
                   jax.ShapeDtypeStruct((B, N, C), _f32), _STAT_SHAPE],
    )(p6, a6, c6, od, p['Ww'][0, :, 0, 0].reshape(C, 1), bw_arr,
      p['Wg'][:, :, 0, 0].T, row(p['bg']))
    a7, c7 = _fold2(st7, N, p['gg'], p['tg'])

    bf_arr = jnp.full((1, 8, C), p['bf'][0], _f32)
    lg = pl.pallas_call(
        _k9,
        grid=(B,),
        in_specs=[_bspec((1, N, C)), _vspec((1, C)), _vspec((1, C)),
                  _bspec((1, N, C)), _vspec((C, 1)), _vspec((1, 8, C))],
        out_specs=_bspec((1, N, 1)),
        out_shape=jax.ShapeDtypeStruct((B, N, 1), _f32),
    )(p7, a7, c7, ex, p['Wf'][0, :, 0, 0].reshape(C, 1), bf_arr)

    return lg[:, :, 0]
